# baseline (speedup 1.0000x reference)
import functools

import jax
import jax.numpy as jnp
from jax import lax
from jax.experimental import pallas as pl
from jax.experimental.pallas import tpu as pltpu
from jax.experimental.pallas import tpu_sc as plsc

_B, _L, _P = 16, 4096, 64
_GRID = 128 * 128 * 128
_N = _B * _L
_NW = 32
_CHUNK = 1024
_NCHUNK = _N // _CHUNK
_PASSES = _NCHUNK // _NW
_BLK = 128


def _combine_body(ps_hbm, perm_hbm, ranks_hbm, feats_hbm, out_hbm,
                  ps_v, perm_v, rank_v, rows_v, acc_v, sem_i, sem_g):
    w = lax.axis_index("s") * 2 + lax.axis_index("c")
    pltpu.sync_copy(ps_hbm.at[w], ps_v)
    pvec = ps_v[pl.ds(0, 16)]
    zero16 = jnp.zeros((16,), jnp.float32)
    iota16 = lax.iota(jnp.int32, 16)

    def fire_idx(pos, par):
        pltpu.async_copy(perm_hbm.at[pl.ds(pos, _BLK)], perm_v[par],
                         sem_i[2 * par])
        pltpu.async_copy(ranks_hbm.at[pl.ds(pos, _BLK)], rank_v[par],
                         sem_i[2 * par + 1])

    def wait_idx(par):
        pltpu.make_async_copy(perm_hbm.at[pl.ds(0, _BLK)], perm_v[par],
                              sem_i[2 * par]).wait()
        pltpu.make_async_copy(ranks_hbm.at[pl.ds(0, _BLK)], rank_v[par],
                              sem_i[2 * par + 1]).wait()

    def fire_gather(par):
        pltpu.async_copy(feats_hbm.at[perm_v[par]], rows_v[par], sem_g[par])

    def wait_gather(par):
        pltpu.make_async_copy(feats_hbm.at[perm_v[par]], rows_v[par],
                              sem_g[par]).wait()

    for q in range(_PASSES):
        r0 = (2 * w + q) * _CHUNK
        p0 = pvec[q]
        p1 = pvec[q + 1]

        def zero_row(i, carry):
            for cg in range(_P // 16):
                acc_v[i, pl.ds(cg * 16, 16)] = zero16
            return carry
        lax.fori_loop(0, _CHUNK, zero_row, 0)

        pa = (p0 // 8) * 8
        nblk = (p1 - pa + _BLK - 1) // _BLK

        def compute(i, par):
            pos = pa + i * _BLK
            rfirst = rank_v[par][pl.ds(0, 16)][0]
            rlast = rank_v[par][pl.ds(_BLK - 16, 16)][15]
            fast = ((pos >= p0) & (pos + _BLK <= p1)
                    & (rlast - rfirst == _BLK - 1))

            @pl.when(fast)
            def _():
                rl0 = rfirst - r0

                def frow(j, c2):
                    for cg in range(_P // 16):
                        plsc.addupdate(
                            acc_v.at[rl0 + j, pl.ds(cg * 16, 16)],
                            rows_v[par][j, pl.ds(cg * 16, 16)])
                    return c2
                lax.fori_loop(0, _BLK, frow, 0)

            @pl.when(jnp.logical_not(fast))
            def _():
                def sgrp(g, c2):
                    pid = pos + g * 16 + iota16
                    ok = (pid >= p0) & (pid < p1)
                    r16 = rank_v[par][pl.ds(g * 16, 16)]
                    rl16 = jnp.where(ok, r16 - r0, _CHUNK)
                    for lane in range(16):
                        rl = rl16[lane]
                        for cg in range(_P // 16):
                            plsc.addupdate(
                                acc_v.at[rl, pl.ds(cg * 16, 16)],
                                rows_v[par][g * 16 + lane,
                                            pl.ds(cg * 16, 16)])
                    return c2
                lax.fori_loop(0, _BLK // 16, sgrp, 0)

        def step(i, par):
            @pl.when(i + 1 < nblk)
            def _():
                wait_idx(1 - par)
                fire_gather(1 - par)
            wait_gather(par)
            compute(i, par)

            @pl.when(i + 2 < nblk)
            def _():
                fire_idx(pa + (i + 2) * _BLK, par)

        @pl.when(nblk > 0)
        def _():
            fire_idx(pa, 0)

            @pl.when(nblk > 1)
            def _():
                fire_idx(pa + _BLK, 1)
            wait_idx(0)
            fire_gather(0)

        def pair(g, carry):
            step(2 * g, 0)

            @pl.when(2 * g + 1 < nblk)
            def _():
                step(2 * g + 1, 1)
            return carry
        lax.fori_loop(0, (nblk + 1) // 2, pair, 0)

        pltpu.sync_copy(acc_v.at[pl.ds(0, _CHUNK)],
                        out_hbm.at[pl.ds(r0, _CHUNK)])


@jax.jit
def kernel(coords, features):
    strides = jnp.array([128 * 128, 128, 1], dtype=jnp.int32)
    lin = (coords.astype(jnp.int32) * strides).sum(-1)
    keys = (lin + jnp.arange(_B, dtype=jnp.int32)[:, None] * _GRID).reshape(-1)
    feats = features.reshape(_N, _P)

    skeys, perm = lax.sort_key_val(keys, jnp.arange(_N, dtype=jnp.int32))
    flags = jnp.concatenate([
        jnp.ones((1,), jnp.int32),
        (skeys[1:] != skeys[:-1]).astype(jnp.int32)])
    ranks = jnp.cumsum(flags) - 1
    bounds = ((jnp.arange(_NW, dtype=jnp.int32)[:, None] * _PASSES
               + jnp.arange(16, dtype=jnp.int32)[None, :]) * _CHUNK)
    ps_rows = jnp.searchsorted(ranks, bounds, side="left").astype(jnp.int32)
    perm_p = jnp.pad(perm, (0, _BLK))
    ranks_p = jnp.pad(ranks, (0, _BLK))

    mesh = plsc.VectorSubcoreMesh(core_axis_name="c", subcore_axis_name="s")
    combine = pl.kernel(
        _combine_body,
        out_type=jax.ShapeDtypeStruct((_N, _P), jnp.float32),
        mesh=mesh,
        scratch_types=[
            pltpu.VMEM((16,), jnp.int32),
            [pltpu.VMEM((_BLK,), jnp.int32) for _ in range(2)],
            [pltpu.VMEM((_BLK,), jnp.int32) for _ in range(2)],
            [pltpu.VMEM((_BLK, _P), jnp.float32) for _ in range(2)],
            pltpu.VMEM((_CHUNK + 1, _P), jnp.float32),
            [pltpu.SemaphoreType.DMA for _ in range(4)],
            [pltpu.SemaphoreType.DMA for _ in range(2)],
        ],
        compiler_params=pltpu.CompilerParams(use_tc_tiling_on_sc=False),
    )
    return combine(ps_rows, perm_p, ranks_p, feats)


# speedup vs baseline: 1.2857x; 1.2857x over previous
import functools

import jax
import jax.numpy as jnp
from jax import lax
from jax.experimental import pallas as pl
from jax.experimental.pallas import tpu as pltpu
from jax.experimental.pallas import tpu_sc as plsc

_B, _L, _P = 16, 4096, 64
_GRID = 128 * 128 * 128
_N = _B * _L
_NW = 32
_CHUNK = 1024
_NCHUNK = _N // _CHUNK
_PASSES = _NCHUNK // _NW
_BLK = 128


def _combine_body(ps_hbm, perm_hbm, ranks_hbm, feats_hbm, out_hbm,
                  ps_v, perm_v, rank_v, rows_v, acc_v, sem_i, sem_g):
    w = lax.axis_index("s") * 2 + lax.axis_index("c")
    pltpu.sync_copy(ps_hbm.at[w], ps_v)
    pvec = ps_v[pl.ds(0, 16)]
    zero16 = jnp.zeros((16,), jnp.float32)
    iota16 = lax.iota(jnp.int32, 16)

    def fire_idx(pos, par):
        pltpu.async_copy(perm_hbm.at[pl.ds(pos, _BLK)], perm_v[par],
                         sem_i[2 * par])
        pltpu.async_copy(ranks_hbm.at[pl.ds(pos, _BLK)], rank_v[par],
                         sem_i[2 * par + 1])

    def wait_idx(par):
        pltpu.make_async_copy(perm_hbm.at[pl.ds(0, _BLK)], perm_v[par],
                              sem_i[2 * par]).wait()
        pltpu.make_async_copy(ranks_hbm.at[pl.ds(0, _BLK)], rank_v[par],
                              sem_i[2 * par + 1]).wait()

    def fire_gather(par):
        pltpu.async_copy(feats_hbm.at[perm_v[par]], rows_v[par], sem_g[par])

    def wait_gather(par):
        pltpu.make_async_copy(feats_hbm.at[perm_v[par]], rows_v[par],
                              sem_g[par]).wait()

    for q in range(_PASSES):
        r0 = (2 * w + q) * _CHUNK
        p0 = pvec[q]
        p1 = pvec[q + 1]

        def zero_row(i, carry):
            for cg in range(_P // 16):
                acc_v[i, pl.ds(cg * 16, 16)] = zero16
            return carry
        lax.fori_loop(0, _CHUNK, zero_row, 0)

        pa = (p0 // 8) * 8
        nblk = (p1 - pa + _BLK - 1) // _BLK

        def compute(i, par):
            pos = pa + i * _BLK
            rfirst = rank_v[par][pl.ds(0, 16)][0]
            rlast = rank_v[par][pl.ds(_BLK - 16, 16)][15]
            fast = ((pos >= p0) & (pos + _BLK <= p1)
                    & (rlast - rfirst == _BLK - 1))

            @pl.when(fast)
            def _():
                rl0 = rfirst - r0

                def frow(j, c2):
                    for cg in range(_P // 16):
                        plsc.addupdate(
                            acc_v.at[rl0 + j, pl.ds(cg * 16, 16)],
                            rows_v[par][j, pl.ds(cg * 16, 16)])
                    return c2
                lax.fori_loop(0, _BLK, frow, 0)

            @pl.when(jnp.logical_not(fast))
            def _():
                def sgrp(g, c2):
                    pid = pos + g * 16 + iota16
                    ok = (pid >= p0) & (pid < p1)
                    r16 = rank_v[par][pl.ds(g * 16, 16)]
                    rl16 = jnp.where(ok, r16 - r0, _CHUNK)
                    for lane in range(16):
                        rl = rl16[lane]
                        for cg in range(_P // 16):
                            plsc.addupdate(
                                acc_v.at[rl, pl.ds(cg * 16, 16)],
                                rows_v[par][g * 16 + lane,
                                            pl.ds(cg * 16, 16)])
                    return c2
                lax.fori_loop(0, _BLK // 16, sgrp, 0)

        def step(i, par):
            @pl.when(i + 1 < nblk)
            def _():
                wait_idx(1 - par)
                fire_gather(1 - par)
            wait_gather(par)
            compute(i, par)

            @pl.when(i + 2 < nblk)
            def _():
                fire_idx(pa + (i + 2) * _BLK, par)

        @pl.when(nblk > 0)
        def _():
            fire_idx(pa, 0)

            @pl.when(nblk > 1)
            def _():
                fire_idx(pa + _BLK, 1)
            wait_idx(0)
            fire_gather(0)

        def pair(g, carry):
            step(2 * g, 0)

            @pl.when(2 * g + 1 < nblk)
            def _():
                step(2 * g + 1, 1)
            return carry
        lax.fori_loop(0, (nblk + 1) // 2, pair, 0)

        pltpu.sync_copy(acc_v.at[pl.ds(0, _CHUNK)],
                        out_hbm.at[pl.ds(r0, _CHUNK)])


@jax.jit
def kernel(coords, features):
    strides = jnp.array([128 * 128, 128, 1], dtype=jnp.int32)
    lin = (coords.astype(jnp.int32) * strides).sum(-1)
    keys = (lin + jnp.arange(_B, dtype=jnp.int32)[:, None] * _GRID).reshape(-1)
    feats = features.reshape(_N, _P)

    skeys, perm = lax.sort_key_val(keys, jnp.arange(_N, dtype=jnp.int32))
    flags = jnp.concatenate([
        jnp.ones((1,), jnp.int32),
        (skeys[1:] != skeys[:-1]).astype(jnp.int32)])
    ranks = jnp.cumsum(flags) - 1
    bounds = jnp.arange(_NCHUNK + 1, dtype=jnp.int32) * _CHUNK
    ps = jnp.sum(ranks[:, None] < bounds[None, :], axis=0, dtype=jnp.int32)
    wi = jnp.arange(_NW)
    ps_rows = jnp.stack([ps[2 * wi], ps[2 * wi + 1], ps[2 * wi + 2]], axis=1)
    ps_rows = jnp.pad(ps_rows, ((0, 0), (0, 16 - _PASSES - 1)))
    perm_p = jnp.pad(perm, (0, _BLK))
    ranks_p = jnp.pad(ranks, (0, _BLK))

    mesh = plsc.VectorSubcoreMesh(core_axis_name="c", subcore_axis_name="s")
    combine = pl.kernel(
        _combine_body,
        out_type=jax.ShapeDtypeStruct((_N, _P), jnp.float32),
        mesh=mesh,
        scratch_types=[
            pltpu.VMEM((16,), jnp.int32),
            [pltpu.VMEM((_BLK,), jnp.int32) for _ in range(2)],
            [pltpu.VMEM((_BLK,), jnp.int32) for _ in range(2)],
            [pltpu.VMEM((_BLK, _P), jnp.float32) for _ in range(2)],
            pltpu.VMEM((_CHUNK + 1, _P), jnp.float32),
            [pltpu.SemaphoreType.DMA for _ in range(4)],
            [pltpu.SemaphoreType.DMA for _ in range(2)],
        ],
        compiler_params=pltpu.CompilerParams(use_tc_tiling_on_sc=False),
    )
    return combine(ps_rows, perm_p, ranks_p, feats)


# speedup vs baseline: 1.3713x; 1.0666x over previous
import functools

import jax
import jax.numpy as jnp
from jax import lax
from jax.experimental import pallas as pl
from jax.experimental.pallas import tpu as pltpu
from jax.experimental.pallas import tpu_sc as plsc

_B, _L, _P = 16, 4096, 64
_GRID = 128 * 128 * 128
_N = _B * _L
_NW = 32
_CHUNK = 1024
_NCHUNK = _N // _CHUNK
_PASSES = _NCHUNK // _NW
_BLK = 128


def _combine_body(ps_hbm, perm_hbm, ranks_hbm, feats_hbm, out_hbm,
                  ps_v, perm_v, rank_v, rows_v, acc_v, sem_i, sem_g):
    w = lax.axis_index("s") * 2 + lax.axis_index("c")
    pltpu.sync_copy(ps_hbm.at[w], ps_v)
    pvec = ps_v[pl.ds(0, 16)]
    zero16 = jnp.zeros((16,), jnp.float32)
    iota16 = lax.iota(jnp.int32, 16)

    def fire_idx(pos, par):
        pltpu.async_copy(perm_hbm.at[pl.ds(pos, _BLK)], perm_v[par],
                         sem_i[2 * par])
        pltpu.async_copy(ranks_hbm.at[pl.ds(pos, _BLK)], rank_v[par],
                         sem_i[2 * par + 1])

    def wait_idx(par):
        pltpu.make_async_copy(perm_hbm.at[pl.ds(0, _BLK)], perm_v[par],
                              sem_i[2 * par]).wait()
        pltpu.make_async_copy(ranks_hbm.at[pl.ds(0, _BLK)], rank_v[par],
                              sem_i[2 * par + 1]).wait()

    def fire_gather(par):
        pltpu.async_copy(feats_hbm.at[perm_v[par]], rows_v[par], sem_g[par])

    def wait_gather(par):
        pltpu.make_async_copy(feats_hbm.at[perm_v[par]], rows_v[par],
                              sem_g[par]).wait()

    for q in range(_PASSES):
        r0 = (2 * w + q) * _CHUNK
        p0 = pvec[q]
        p1 = pvec[q + 1]

        def zero_row(i, carry):
            for cg in range(_P // 16):
                acc_v[i, pl.ds(cg * 16, 16)] = zero16
            return carry
        lax.fori_loop(0, _CHUNK, zero_row, 0)

        pa = (p0 // 8) * 8
        nblk = (p1 - pa + _BLK - 1) // _BLK

        def compute(i, par):
            pos = pa + i * _BLK
            rfirst = rank_v[par][pl.ds(0, 16)][0]
            rlast = rank_v[par][pl.ds(_BLK - 16, 16)][15]
            fast = ((pos >= p0) & (pos + _BLK <= p1)
                    & (rlast - rfirst == _BLK - 1))

            @pl.when(fast)
            def _():
                rl0 = rfirst - r0

                def frow(j, c2):
                    for cg in range(_P // 16):
                        plsc.addupdate(
                            acc_v.at[rl0 + j, pl.ds(cg * 16, 16)],
                            rows_v[par][j, pl.ds(cg * 16, 16)])
                    return c2
                lax.fori_loop(0, _BLK, frow, 0)

            @pl.when(jnp.logical_not(fast))
            def _():
                def sgrp(g, c2):
                    pid = pos + g * 16 + iota16
                    ok = (pid >= p0) & (pid < p1)
                    r16 = rank_v[par][pl.ds(g * 16, 16)]
                    rl16 = jnp.where(ok, r16 - r0, _CHUNK)
                    for lane in range(16):
                        rl = rl16[lane]
                        for cg in range(_P // 16):
                            plsc.addupdate(
                                acc_v.at[rl, pl.ds(cg * 16, 16)],
                                rows_v[par][g * 16 + lane,
                                            pl.ds(cg * 16, 16)])
                    return c2
                lax.fori_loop(0, _BLK // 16, sgrp, 0)

        def step(i, par):
            @pl.when(i + 1 < nblk)
            def _():
                wait_idx(1 - par)
                fire_gather(1 - par)
            wait_gather(par)
            compute(i, par)

            @pl.when(i + 2 < nblk)
            def _():
                fire_idx(pa + (i + 2) * _BLK, par)

        @pl.when(nblk > 0)
        def _():
            fire_idx(pa, 0)

            @pl.when(nblk > 1)
            def _():
                fire_idx(pa + _BLK, 1)
            wait_idx(0)
            fire_gather(0)

        def pair(g, carry):
            step(2 * g, 0)

            @pl.when(2 * g + 1 < nblk)
            def _():
                step(2 * g + 1, 1)
            return carry
        lax.fori_loop(0, (nblk + 1) // 2, pair, 0)

        pltpu.sync_copy(acc_v.at[pl.ds(0, _CHUNK)],
                        out_hbm.at[pl.ds(r0, _CHUNK)])


@jax.jit
def kernel(coords, features):
    strides = jnp.array([128 * 128, 128, 1], dtype=jnp.int32)
    lin = (coords.astype(jnp.int32) * strides).sum(-1)
    feats = features.reshape(_N, _P)

    vals = jnp.broadcast_to(jnp.arange(_L, dtype=jnp.int32)[None, :],
                            (_B, _L))
    slin, sval = lax.sort_key_val(lin, vals)
    boff = jnp.arange(_B, dtype=jnp.int32)[:, None]
    skeys = (slin + boff * _GRID).reshape(-1)
    perm = (sval + boff * _L).reshape(-1)
    flags = jnp.concatenate([
        jnp.ones((1,), jnp.int32),
        (skeys[1:] != skeys[:-1]).astype(jnp.int32)])
    ranks = jnp.cumsum(flags) - 1
    bounds = jnp.arange(_NCHUNK + 1, dtype=jnp.int32) * _CHUNK
    ps = jnp.sum(ranks[:, None] < bounds[None, :], axis=0, dtype=jnp.int32)
    wi = jnp.arange(_NW)
    ps_rows = jnp.stack([ps[2 * wi], ps[2 * wi + 1], ps[2 * wi + 2]], axis=1)
    ps_rows = jnp.pad(ps_rows, ((0, 0), (0, 16 - _PASSES - 1)))
    perm_p = jnp.pad(perm, (0, _BLK))
    ranks_p = jnp.pad(ranks, (0, _BLK))

    mesh = plsc.VectorSubcoreMesh(core_axis_name="c", subcore_axis_name="s")
    combine = pl.kernel(
        _combine_body,
        out_type=jax.ShapeDtypeStruct((_N, _P), jnp.float32),
        mesh=mesh,
        scratch_types=[
            pltpu.VMEM((16,), jnp.int32),
            [pltpu.VMEM((_BLK,), jnp.int32) for _ in range(2)],
            [pltpu.VMEM((_BLK,), jnp.int32) for _ in range(2)],
            [pltpu.VMEM((_BLK, _P), jnp.float32) for _ in range(2)],
            pltpu.VMEM((_CHUNK + 1, _P), jnp.float32),
            [pltpu.SemaphoreType.DMA for _ in range(4)],
            [pltpu.SemaphoreType.DMA for _ in range(2)],
        ],
        compiler_params=pltpu.CompilerParams(use_tc_tiling_on_sc=False),
    )
    return combine(ps_rows, perm_p, ranks_p, feats)


# speedup vs baseline: 1.4109x; 1.0289x over previous
import functools

import jax
import jax.numpy as jnp
from jax import lax
from jax.experimental import pallas as pl
from jax.experimental.pallas import tpu as pltpu
from jax.experimental.pallas import tpu_sc as plsc

_B, _L, _P = 16, 4096, 64
_GRID = 128 * 128 * 128
_N = _B * _L
_NW = 32
_CHUNK = 512
_NCHUNK = _N // _CHUNK
_PASSES = _NCHUNK // _NW
_BLK = 128


def _combine_body(ps_hbm, perm_hbm, ranks_hbm, feats_hbm, out_hbm,
                  ps_v, perm_v, rank_v, rows_v, acc_v, sem_i, sem_g):
    w = lax.axis_index("s") * 2 + lax.axis_index("c")
    pltpu.sync_copy(ps_hbm.at[w], ps_v)
    pvec = ps_v[pl.ds(0, 16)]
    zero16 = jnp.zeros((16,), jnp.float32)
    iota16 = lax.iota(jnp.int32, 16)

    def fire_idx(pos, par):
        pltpu.async_copy(perm_hbm.at[pl.ds(pos, _BLK)], perm_v[par],
                         sem_i[2 * par])
        pltpu.async_copy(ranks_hbm.at[pl.ds(pos, _BLK)], rank_v[par],
                         sem_i[2 * par + 1])

    def wait_idx(par):
        pltpu.make_async_copy(perm_hbm.at[pl.ds(0, _BLK)], perm_v[par],
                              sem_i[2 * par]).wait()
        pltpu.make_async_copy(ranks_hbm.at[pl.ds(0, _BLK)], rank_v[par],
                              sem_i[2 * par + 1]).wait()

    def fire_gather(par):
        pltpu.async_copy(feats_hbm.at[perm_v[par]], rows_v[par], sem_g[par])

    def wait_gather(par):
        pltpu.make_async_copy(feats_hbm.at[perm_v[par]], rows_v[par],
                              sem_g[par]).wait()

    for q in range(_PASSES):
        r0 = (_PASSES * w + q) * _CHUNK
        p0 = pvec[q]
        p1 = pvec[q + 1]

        def zero_row(i, carry):
            for cg in range(_P // 16):
                acc_v[i, pl.ds(cg * 16, 16)] = zero16
            return carry
        lax.fori_loop(0, _CHUNK, zero_row, 0)

        pa = (p0 // 8) * 8
        nblk = (p1 - pa + _BLK - 1) // _BLK

        def compute(i, par):
            pos = pa + i * _BLK
            rfirst = rank_v[par][pl.ds(0, 16)][0]
            rlast = rank_v[par][pl.ds(_BLK - 16, 16)][15]
            fast = ((pos >= p0) & (pos + _BLK <= p1)
                    & (rlast - rfirst == _BLK - 1))

            @pl.when(fast)
            def _():
                rl0 = rfirst - r0

                def frow(j, c2):
                    for cg in range(_P // 16):
                        plsc.addupdate(
                            acc_v.at[rl0 + j, pl.ds(cg * 16, 16)],
                            rows_v[par][j, pl.ds(cg * 16, 16)])
                    return c2
                lax.fori_loop(0, _BLK, frow, 0)

            @pl.when(jnp.logical_not(fast))
            def _():
                def sgrp(g, c2):
                    pid = pos + g * 16 + iota16
                    ok = (pid >= p0) & (pid < p1)
                    r16 = rank_v[par][pl.ds(g * 16, 16)]
                    rl16 = jnp.where(ok, r16 - r0, _CHUNK)
                    for lane in range(16):
                        rl = rl16[lane]
                        for cg in range(_P // 16):
                            plsc.addupdate(
                                acc_v.at[rl, pl.ds(cg * 16, 16)],
                                rows_v[par][g * 16 + lane,
                                            pl.ds(cg * 16, 16)])
                    return c2
                lax.fori_loop(0, _BLK // 16, sgrp, 0)

        def step(i, par):
            @pl.when(i + 1 < nblk)
            def _():
                wait_idx(1 - par)
                fire_gather(1 - par)
            wait_gather(par)
            compute(i, par)

            @pl.when(i + 2 < nblk)
            def _():
                fire_idx(pa + (i + 2) * _BLK, par)

        @pl.when(nblk > 0)
        def _():
            fire_idx(pa, 0)

            @pl.when(nblk > 1)
            def _():
                fire_idx(pa + _BLK, 1)
            wait_idx(0)
            fire_gather(0)

        def pair(g, carry):
            step(2 * g, 0)

            @pl.when(2 * g + 1 < nblk)
            def _():
                step(2 * g + 1, 1)
            return carry
        lax.fori_loop(0, (nblk + 1) // 2, pair, 0)

        pltpu.sync_copy(acc_v.at[pl.ds(0, _CHUNK)],
                        out_hbm.at[pl.ds(r0, _CHUNK)])


@jax.jit
def kernel(coords, features):
    strides = jnp.array([128 * 128, 128, 1], dtype=jnp.int32)
    lin = (coords.astype(jnp.int32) * strides).sum(-1)
    feats = jnp.pad(features.reshape(_N, _P), ((0, 0), (0, 128 - _P)))

    vals = jnp.broadcast_to(jnp.arange(_L, dtype=jnp.int32)[None, :],
                            (_B, _L))
    slin, sval = lax.sort_key_val(lin, vals)
    boff = jnp.arange(_B, dtype=jnp.int32)[:, None]
    skeys = (slin + boff * _GRID).reshape(-1)
    perm = (sval + boff * _L).reshape(-1)
    flags = jnp.concatenate([
        jnp.ones((1,), jnp.int32),
        (skeys[1:] != skeys[:-1]).astype(jnp.int32)])
    ranks = jnp.cumsum(flags) - 1
    bounds = jnp.arange(_NCHUNK + 1, dtype=jnp.int32) * _CHUNK
    ps = jnp.sum(ranks[:, None] < bounds[None, :], axis=0, dtype=jnp.int32)
    wi = jnp.arange(_NW)
    ps_rows = jnp.stack(
        [ps[_PASSES * wi + j] for j in range(_PASSES + 1)], axis=1)
    ps_rows = jnp.pad(ps_rows, ((0, 0), (0, 16 - _PASSES - 1)))
    perm_p = jnp.pad(perm, (0, _BLK))
    ranks_p = jnp.pad(ranks, (0, _BLK))

    mesh = plsc.VectorSubcoreMesh(core_axis_name="c", subcore_axis_name="s")
    combine = pl.kernel(
        _combine_body,
        out_type=jax.ShapeDtypeStruct((_N, _P), jnp.float32),
        mesh=mesh,
        scratch_types=[
            pltpu.VMEM((16,), jnp.int32),
            [pltpu.VMEM((_BLK,), jnp.int32) for _ in range(2)],
            [pltpu.VMEM((_BLK,), jnp.int32) for _ in range(2)],
            [pltpu.VMEM((_BLK, 128), jnp.float32) for _ in range(2)],
            pltpu.VMEM((_CHUNK + 1, _P), jnp.float32),
            [pltpu.SemaphoreType.DMA for _ in range(4)],
            [pltpu.SemaphoreType.DMA for _ in range(2)],
        ],
    )
    return combine(ps_rows, perm_p, ranks_p, feats)
